# baseline (speedup 1.0000x reference)
import functools

import jax
import jax.numpy as jnp
from jax import lax
from jax.experimental import pallas as pl
from jax.experimental.pallas import tpu as pltpu
from jax.experimental.pallas import tpu_sc as plsc

N_NODES_C = 50000
N_EDGES_C = 800000
LATENT_C = 64

_NC = 2
_NS = 16
_NW = _NC * _NS

_IDXW = 125


@functools.lru_cache(maxsize=None)
def _sc_mesh():
    return plsc.VectorSubcoreMesh(core_axis_name="c", subcore_axis_name="s",
                                  num_cores=_NC, num_subcores=_NS)


_G = 250


@functools.lru_cache(maxsize=None)
def _sc_gather_kernel(n_edges):
    e_w = n_edges // _NW
    ng = e_w // _G
    nrow = e_w // _IDXW

    def body(x_hbm, src_hbm, dst_hbm, g2_hbm,
             sidx, didx, srows, drows, gsem, dsem, osem_s, osem_r):
        wid = lax.axis_index("s") * _NC + lax.axis_index("c")
        e0 = wid * e_w
        pltpu.sync_copy(src_hbm.at[pl.ds(wid * nrow, nrow)], sidx)
        pltpu.sync_copy(dst_hbm.at[pl.ds(wid * nrow, nrow)], didx)

        def fire(g, b):
            for j in range(_G // _IDXW):
                kk = g * (_G // _IDXW) + j
                pltpu.async_copy(x_hbm.at[sidx.at[kk]],
                                 srows[b].at[pl.ds(j * _IDXW, _IDXW)],
                                 gsem[b])
                pltpu.async_copy(x_hbm.at[didx.at[kk]],
                                 drows[b].at[pl.ds(j * _IDXW, _IDXW)],
                                 dsem[b])

        def drain_gathers(b):
            for j in range(_G // _IDXW):
                pltpu.make_async_copy(
                    x_hbm.at[sidx.at[0]],
                    srows[b].at[pl.ds(0, _IDXW)], gsem[b]).wait()
                pltpu.make_async_copy(
                    x_hbm.at[didx.at[0]],
                    drows[b].at[pl.ds(0, _IDXW)], dsem[b]).wait()

        def writeback(g, b):
            base = e0 + g * _G
            pltpu.async_copy(
                srows[b], g2_hbm.at[pl.ds(base, _G), pl.ds(0, LATENT_C)],
                osem_s[b])
            pltpu.async_copy(
                drows[b],
                g2_hbm.at[pl.ds(base, _G), pl.ds(LATENT_C, LATENT_C)],
                osem_r[b])

        def wait_writeback(b):
            pltpu.make_async_copy(
                srows[b], g2_hbm.at[pl.ds(0, _G), pl.ds(0, LATENT_C)],
                osem_s[b]).wait()
            pltpu.make_async_copy(
                drows[b], g2_hbm.at[pl.ds(0, _G), pl.ds(LATENT_C, LATENT_C)],
                osem_r[b]).wait()

        def loop(i, _):
            for b in range(2):
                g = 2 * i + b

                @pl.when(g >= 2)
                def _():
                    wait_writeback(b)

                fire(g, b)

                @pl.when(g >= 1)
                def _():
                    drain_gathers(1 - b)
                    writeback(g - 1, 1 - b)

            return ()

        lax.fori_loop(0, ng // 2, loop, ())
        drain_gathers(1)
        writeback(ng - 1, 1)
        wait_writeback(0)
        wait_writeback(1)

    return functools.partial(
        pl.kernel,
        out_type=jax.ShapeDtypeStruct((n_edges, 2 * LATENT_C), jnp.float32),
        mesh=_sc_mesh(),
        scratch_types=[
            pltpu.VMEM((nrow, _IDXW), jnp.int32),
            pltpu.VMEM((nrow, _IDXW), jnp.int32),
            [pltpu.VMEM((_G, LATENT_C), jnp.float32) for _ in range(2)],
            [pltpu.VMEM((_G, LATENT_C), jnp.float32) for _ in range(2)],
            [pltpu.SemaphoreType.DMA for _ in range(2)],
            [pltpu.SemaphoreType.DMA for _ in range(2)],
            [pltpu.SemaphoreType.DMA for _ in range(2)],
            [pltpu.SemaphoreType.DMA for _ in range(2)],
        ],
        compiler_params=pltpu.CompilerParams(use_tc_tiling_on_sc=False),
    )(body)


_COLS = LATENT_C // _NC
_G2 = 250
_RPT = N_NODES_C // _NS
_GIR = _G2 // _IDXW


@functools.lru_cache(maxsize=None)
def _sc_scatter_kernel(n_edges):
    e_t = n_edges // _NS
    ng2 = e_t // _G2

    def body(upd_hbm, dst_hbm, agg_hbm, idx2, rows, rsem, isem, acc_sh):
        c = lax.axis_index("c")
        s = lax.axis_index("s")
        col0 = c * _COLS
        r0 = s * _RPT
        e0 = s * e_t
        i0 = s * (e_t // _IDXW)

        zero16 = jnp.zeros((16,), jnp.float32)

        def zfill(i, _):
            rows[0][i // 2, pl.ds((i % 2) * 16, 16)] = zero16
            return ()

        lax.fori_loop(0, _G2 * 2, zfill, ())

        nfull = _RPT // _G2
        rem = _RPT - nfull * _G2

        def zcopy(kk, _):
            pltpu.sync_copy(rows[0], acc_sh.at[pl.ds(r0 + kk * _G2, _G2)])
            return ()

        lax.fori_loop(0, nfull, zcopy, ())
        pltpu.sync_copy(rows[0].at[pl.ds(0, rem)],
                        acc_sh.at[pl.ds(r0 + nfull * _G2, rem)])
        plsc.subcore_barrier()

        def fire(g, b):
            base = e0 + g * _G2
            pltpu.async_copy(dst_hbm.at[pl.ds(i0 + g * _GIR, _GIR)],
                             idx2[b], isem[b])
            pltpu.async_copy(
                upd_hbm.at[pl.ds(base, _G2), pl.ds(col0, _COLS)],
                rows[b], rsem[b])

        def complete(b):
            pltpu.make_async_copy(dst_hbm.at[pl.ds(0, _GIR)],
                                  idx2[b], isem[b]).wait()
            pltpu.make_async_copy(
                upd_hbm.at[pl.ds(0, _G2), pl.ds(col0, _COLS)],
                rows[b], rsem[b]).wait()
            for j in range(_GIR):
                pltpu.sync_copy(rows[b].at[pl.ds(j * _IDXW, _IDXW)],
                                acc_sh.at[idx2[b].at[j]], add=True)

        def loop(i, _):
            for b in range(2):
                g = 2 * i + b
                fire(g, b)

                @pl.when(g >= 1)
                def _():
                    complete(1 - b)

            return ()

        lax.fori_loop(0, ng2 // 2, loop, ())
        complete(1)
        plsc.subcore_barrier()

        pltpu.sync_copy(acc_sh.at[pl.ds(r0, _RPT)],
                        agg_hbm.at[pl.ds(r0, _RPT), pl.ds(col0, _COLS)])

    return functools.partial(
        pl.kernel,
        out_type=jax.ShapeDtypeStruct((N_NODES_C, LATENT_C), jnp.float32),
        mesh=_sc_mesh(),
        scratch_types=[
            [pltpu.VMEM((_GIR, _IDXW), jnp.int32) for _ in range(2)],
            [pltpu.VMEM((_G2, _COLS), jnp.float32) for _ in range(2)],
            [pltpu.SemaphoreType.DMA for _ in range(2)],
            [pltpu.SemaphoreType.DMA for _ in range(2)],
            pltpu.VMEM_SHARED((N_NODES_C, _COLS), jnp.float32),
        ],
        compiler_params=pltpu.CompilerParams(use_tc_tiling_on_sc=False),
    )(body)


_EBLK = 10000


def _bdot(a, b):
    return jnp.dot(a, b, preferred_element_type=jnp.float32)


def _edge_core(g2b, e, w1g_ref, w1e_ref, b1_ref, w2_ref, b2_ref,
               g_ref, bb_ref, last):
    h = _bdot(g2b, w1g_ref[...])
    h = h + _bdot(e, w1e_ref[...])
    h = h + b1_ref[...]
    h = h * jax.nn.sigmoid(h)
    o = _bdot(h, w2_ref[...])
    o = o + b2_ref[...]
    mu = jnp.mean(o, axis=-1, keepdims=True)
    var = jnp.mean((o - mu) ** 2, axis=-1, keepdims=True)
    on = (o - mu) * lax.rsqrt(var + 1e-5)
    upd = on * g_ref[...] + bb_ref[...]
    if last:
        return upd, e + upd
    return jnp.concatenate([upd, e + upd], axis=-1), None


def _edge_mlp_first_body(g2_ref, e_ref, w1g_ref, w1e_ref, b1_ref, w2_ref,
                         b2_ref, g_ref, bb_ref, upd2_ref):
    upd2_ref[...], _ = _edge_core(g2_ref[...], e_ref[...], w1g_ref, w1e_ref,
                                  b1_ref, w2_ref, b2_ref, g_ref, bb_ref,
                                  last=False)


def _edge_mlp_mid_body(g2_ref, p2_ref, w1g_ref, w1e_ref, b1_ref, w2_ref,
                       b2_ref, g_ref, bb_ref, upd2_ref):
    upd2_ref[...], _ = _edge_core(g2_ref[...], p2_ref[:, LATENT_C:],
                                  w1g_ref, w1e_ref, b1_ref, w2_ref, b2_ref,
                                  g_ref, bb_ref, last=False)


def _edge_mlp_last_body(g2_ref, p2_ref, w1g_ref, w1e_ref, b1_ref, w2_ref,
                        b2_ref, g_ref, bb_ref, upd2_ref, enew_ref):
    upd, enew = _edge_core(g2_ref[...], p2_ref[:, LATENT_C:],
                           w1g_ref, w1e_ref, b1_ref, w2_ref, b2_ref,
                           g_ref, bb_ref, last=True)
    upd2_ref[...] = jnp.concatenate([upd, upd], axis=-1)
    enew_ref[...] = enew


def _tc_edge_mlp(kind, g2, e, w1, b1, w2, b2, g, b):
    n_e = g2.shape[0]
    grid = (n_e // _EBLK,)
    wide = pl.BlockSpec((_EBLK, 2 * LATENT_C), lambda i: (i, 0))
    if kind == "first":
        e_spec = pl.BlockSpec((_EBLK, LATENT_C), lambda i: (i, 0))
    else:
        e_spec = wide
    full = lambda a: pl.BlockSpec(a.shape, lambda i: (0,) * a.ndim)
    body = {"first": _edge_mlp_first_body, "mid": _edge_mlp_mid_body,
            "last": _edge_mlp_last_body}[kind]
    wide_out = jax.ShapeDtypeStruct((n_e, 2 * LATENT_C), jnp.float32)
    if kind == "last":
        out_specs = [wide, pl.BlockSpec((_EBLK, LATENT_C), lambda i: (i, 0))]
        out_shape = [wide_out,
                     jax.ShapeDtypeStruct((n_e, LATENT_C), jnp.float32)]
    else:
        out_specs = wide
        out_shape = wide_out
    w1g = w1[:2 * LATENT_C]
    w1e = w1[2 * LATENT_C:]
    return pl.pallas_call(
        body,
        grid=grid,
        in_specs=[wide, e_spec,
                  full(w1g), full(w1e), full(b1), full(w2), full(b2),
                  full(g), full(b)],
        out_specs=out_specs,
        out_shape=out_shape,
    )(g2, e, w1g, w1e, b1, w2, b2, g, b)


_NBLK = 5000


def _node_mlp_body(x_ref, a_ref, w1x_ref, w1a_ref, b1_ref, w2_ref, b2_ref,
                   g_ref, bb_ref, xnew_ref):
    h = _bdot(x_ref[...], w1x_ref[...])
    h = h + _bdot(a_ref[...], w1a_ref[...])
    h = h + b1_ref[...]
    h = h * jax.nn.sigmoid(h)
    o = _bdot(h, w2_ref[...])
    o = o + b2_ref[...]
    mu = jnp.mean(o, axis=-1, keepdims=True)
    var = jnp.mean((o - mu) ** 2, axis=-1, keepdims=True)
    on = (o - mu) * lax.rsqrt(var + 1e-5)
    xnew_ref[...] = x_ref[...] + on * g_ref[...] + bb_ref[...]


def _tc_node_mlp(x, agg, w1, b1, w2, b2, g, b):
    grid = (N_NODES_C // _NBLK,)
    row_spec = pl.BlockSpec((_NBLK, LATENT_C), lambda i: (i, 0))
    full = lambda a: pl.BlockSpec(a.shape, lambda i: (0,) * a.ndim)
    w1x = w1[:LATENT_C]
    w1a = w1[LATENT_C:]
    return pl.pallas_call(
        _node_mlp_body,
        grid=grid,
        in_specs=[row_spec, row_spec,
                  full(w1x), full(w1a), full(b1), full(w2), full(b2),
                  full(g), full(b)],
        out_specs=row_spec,
        out_shape=jax.ShapeDtypeStruct((N_NODES_C, LATENT_C), jnp.float32),
    )(x, agg, w1x, w1a, b1, w2, b2, g, b)


def kernel(x, edge_index, edge_attr, params):
    src2 = edge_index[0].astype(jnp.int32).reshape(N_EDGES_C // _IDXW, _IDXW)
    dst2 = edge_index[1].astype(jnp.int32).reshape(N_EDGES_C // _IDXW, _IDXW)
    row2 = lambda a: a.reshape(1, -1)
    n_layers = len(params)
    prev2 = None
    e_new = None
    for li, lp in enumerate(params):
        ep, np_ = lp['edge'], lp['node']
        kind = "first" if li == 0 else "mid"
        g2 = _sc_gather_kernel(N_EDGES_C)(x, src2, dst2)
        e_arg = edge_attr if li == 0 else prev2
        upd2 = _tc_edge_mlp(kind, g2, e_arg,
                            ep['W1'], row2(ep['b1']),
                            ep['W2'], row2(ep['b2']),
                            row2(ep['g']), row2(ep['b']))
        agg = _sc_scatter_kernel(N_EDGES_C)(upd2, dst2)
        x = _tc_node_mlp(x, agg,
                         np_['W1'], row2(np_['b1']),
                         np_['W2'], row2(np_['b2']),
                         row2(np_['g']), row2(np_['b']))
        prev2 = upd2
    return (x, prev2[:, LATENT_C:])


# speedup vs baseline: 1.0449x; 1.0449x over previous
import functools

import jax
import jax.numpy as jnp
from jax import lax
from jax.experimental import pallas as pl
from jax.experimental.pallas import tpu as pltpu
from jax.experimental.pallas import tpu_sc as plsc

N_NODES_C = 50000
N_EDGES_C = 800000
LATENT_C = 64

_NC = 2
_NS = 16
_NW = _NC * _NS

_IDXW = 125


@functools.lru_cache(maxsize=None)
def _sc_mesh():
    return plsc.VectorSubcoreMesh(core_axis_name="c", subcore_axis_name="s",
                                  num_cores=_NC, num_subcores=_NS)


_G = 250


@functools.lru_cache(maxsize=None)
def _sc_gather_kernel(n_edges):
    e_w = n_edges // _NW
    ng = e_w // _G
    nrow = e_w // _IDXW

    def body(x_hbm, src_hbm, dst_hbm, g2_hbm,
             sidx, didx, srows, drows, gsem, dsem, osem_s, osem_r):
        wid = lax.axis_index("s") * _NC + lax.axis_index("c")
        e0 = wid * e_w
        pltpu.sync_copy(src_hbm.at[pl.ds(wid * nrow, nrow)], sidx)
        pltpu.sync_copy(dst_hbm.at[pl.ds(wid * nrow, nrow)], didx)

        def fire(g, b):
            for j in range(_G // _IDXW):
                kk = g * (_G // _IDXW) + j
                pltpu.async_copy(x_hbm.at[sidx.at[kk]],
                                 srows[b].at[pl.ds(j * _IDXW, _IDXW)],
                                 gsem[b])
                pltpu.async_copy(x_hbm.at[didx.at[kk]],
                                 drows[b].at[pl.ds(j * _IDXW, _IDXW)],
                                 dsem[b])

        def drain_gathers(b):
            for j in range(_G // _IDXW):
                pltpu.make_async_copy(
                    x_hbm.at[sidx.at[0]],
                    srows[b].at[pl.ds(0, _IDXW)], gsem[b]).wait()
                pltpu.make_async_copy(
                    x_hbm.at[didx.at[0]],
                    drows[b].at[pl.ds(0, _IDXW)], dsem[b]).wait()

        def writeback(g, b):
            base = e0 + g * _G
            pltpu.async_copy(
                srows[b], g2_hbm.at[pl.ds(base, _G), pl.ds(0, LATENT_C)],
                osem_s[b])
            pltpu.async_copy(
                drows[b],
                g2_hbm.at[pl.ds(base, _G), pl.ds(LATENT_C, LATENT_C)],
                osem_r[b])

        def wait_writeback(b):
            pltpu.make_async_copy(
                srows[b], g2_hbm.at[pl.ds(0, _G), pl.ds(0, LATENT_C)],
                osem_s[b]).wait()
            pltpu.make_async_copy(
                drows[b], g2_hbm.at[pl.ds(0, _G), pl.ds(LATENT_C, LATENT_C)],
                osem_r[b]).wait()

        def loop(i, _):
            for b in range(2):
                g = 2 * i + b

                @pl.when(g >= 2)
                def _():
                    wait_writeback(b)

                fire(g, b)

                @pl.when(g >= 1)
                def _():
                    drain_gathers(1 - b)
                    writeback(g - 1, 1 - b)

            return ()

        lax.fori_loop(0, ng // 2, loop, ())
        drain_gathers(1)
        writeback(ng - 1, 1)
        wait_writeback(0)
        wait_writeback(1)

    return functools.partial(
        pl.kernel,
        out_type=jax.ShapeDtypeStruct((n_edges, 2 * LATENT_C), jnp.float32),
        mesh=_sc_mesh(),
        scratch_types=[
            pltpu.VMEM((nrow, _IDXW), jnp.int32),
            pltpu.VMEM((nrow, _IDXW), jnp.int32),
            [pltpu.VMEM((_G, LATENT_C), jnp.float32) for _ in range(2)],
            [pltpu.VMEM((_G, LATENT_C), jnp.float32) for _ in range(2)],
            [pltpu.SemaphoreType.DMA for _ in range(2)],
            [pltpu.SemaphoreType.DMA for _ in range(2)],
            [pltpu.SemaphoreType.DMA for _ in range(2)],
            [pltpu.SemaphoreType.DMA for _ in range(2)],
        ],
        compiler_params=pltpu.CompilerParams(use_tc_tiling_on_sc=False),
    )(body)


_COLS = LATENT_C // _NC
_G2 = 250
_RPT = N_NODES_C // _NS
_GIR = _G2 // _IDXW


@functools.lru_cache(maxsize=None)
def _sc_scatter_kernel(n_edges):
    e_t = n_edges // _NS
    ng2 = e_t // _G2

    def body(upd_hbm, dst_hbm, agg_hbm, idx2, rows, rsem, isem, acc_sh):
        c = lax.axis_index("c")
        s = lax.axis_index("s")
        col0 = c * _COLS
        r0 = s * _RPT
        e0 = s * e_t
        i0 = s * (e_t // _IDXW)

        zero16 = jnp.zeros((16,), jnp.float32)

        def zfill(i, _):
            rows[0][i // 2, pl.ds((i % 2) * 16, 16)] = zero16
            return ()

        lax.fori_loop(0, _G2 * 2, zfill, ())

        nfull = _RPT // _G2
        rem = _RPT - nfull * _G2

        def zcopy(kk, _):
            pltpu.sync_copy(rows[0], acc_sh.at[pl.ds(r0 + kk * _G2, _G2)])
            return ()

        lax.fori_loop(0, nfull, zcopy, ())
        pltpu.sync_copy(rows[0].at[pl.ds(0, rem)],
                        acc_sh.at[pl.ds(r0 + nfull * _G2, rem)])
        plsc.subcore_barrier()

        def fire(g, b):
            base = e0 + g * _G2
            pltpu.async_copy(dst_hbm.at[pl.ds(i0 + g * _GIR, _GIR)],
                             idx2[b], isem[b])
            pltpu.async_copy(
                upd_hbm.at[pl.ds(base, _G2), pl.ds(col0, _COLS)],
                rows[b], rsem[b])

        def complete(b):
            pltpu.make_async_copy(dst_hbm.at[pl.ds(0, _GIR)],
                                  idx2[b], isem[b]).wait()
            pltpu.make_async_copy(
                upd_hbm.at[pl.ds(0, _G2), pl.ds(col0, _COLS)],
                rows[b], rsem[b]).wait()
            for j in range(_GIR):
                pltpu.sync_copy(rows[b].at[pl.ds(j * _IDXW, _IDXW)],
                                acc_sh.at[idx2[b].at[j]], add=True)

        def loop(i, _):
            for b in range(2):
                g = 2 * i + b
                fire(g, b)

                @pl.when(g >= 1)
                def _():
                    complete(1 - b)

            return ()

        lax.fori_loop(0, ng2 // 2, loop, ())
        complete(1)
        plsc.subcore_barrier()

        pltpu.sync_copy(acc_sh.at[pl.ds(r0, _RPT)],
                        agg_hbm.at[pl.ds(r0, _RPT), pl.ds(col0, _COLS)])

    return functools.partial(
        pl.kernel,
        out_type=jax.ShapeDtypeStruct((N_NODES_C, LATENT_C), jnp.float32),
        mesh=_sc_mesh(),
        scratch_types=[
            [pltpu.VMEM((_GIR, _IDXW), jnp.int32) for _ in range(2)],
            [pltpu.VMEM((_G2, _COLS), jnp.float32) for _ in range(2)],
            [pltpu.SemaphoreType.DMA for _ in range(2)],
            [pltpu.SemaphoreType.DMA for _ in range(2)],
            pltpu.VMEM_SHARED((N_NODES_C, _COLS), jnp.float32),
        ],
        compiler_params=pltpu.CompilerParams(use_tc_tiling_on_sc=False),
    )(body)


_EBLK = 6400


def _bdot(a, b):
    return jnp.dot(a, b, preferred_element_type=jnp.float32)


def _edge_core(g2b, e, w1g_ref, w1e_ref, b1_ref, w2_ref, b2_ref,
               g_ref, bb_ref, last):
    h = _bdot(g2b, w1g_ref[...])
    h = h + _bdot(e, w1e_ref[...])
    h = h + b1_ref[...]
    h = h * jax.nn.sigmoid(h)
    o = _bdot(h, w2_ref[...])
    o = o + b2_ref[...]
    mu = jnp.mean(o, axis=-1, keepdims=True)
    var = jnp.mean((o - mu) ** 2, axis=-1, keepdims=True)
    on = (o - mu) * lax.rsqrt(var + 1e-5)
    upd = on * g_ref[...] + bb_ref[...]
    if last:
        return upd, e + upd
    return jnp.concatenate([upd, e + upd], axis=-1), None


def _edge_mlp_first_body(g2_ref, e_ref, w1g_ref, w1e_ref, b1_ref, w2_ref,
                         b2_ref, g_ref, bb_ref, upd2_ref):
    e = e_ref[...].T
    upd2_ref[...], _ = _edge_core(g2_ref[...], e, w1g_ref, w1e_ref,
                                  b1_ref, w2_ref, b2_ref, g_ref, bb_ref,
                                  last=False)


def _edge_mlp_mid_body(g2_ref, p2_ref, w1g_ref, w1e_ref, b1_ref, w2_ref,
                       b2_ref, g_ref, bb_ref, upd2_ref):
    upd2_ref[...], _ = _edge_core(g2_ref[...], p2_ref[:, LATENT_C:],
                                  w1g_ref, w1e_ref, b1_ref, w2_ref, b2_ref,
                                  g_ref, bb_ref, last=False)


def _edge_mlp_last_body(g2_ref, p2_ref, w1g_ref, w1e_ref, b1_ref, w2_ref,
                        b2_ref, g_ref, bb_ref, upd2_ref, enew_ref):
    upd, enew = _edge_core(g2_ref[...], p2_ref[:, LATENT_C:],
                           w1g_ref, w1e_ref, b1_ref, w2_ref, b2_ref,
                           g_ref, bb_ref, last=True)
    upd2_ref[...] = jnp.concatenate([upd, upd], axis=-1)
    enew_ref[...] = enew.T


def _tc_edge_mlp(kind, g2, e, w1, b1, w2, b2, g, b):
    n_e = g2.shape[0]
    grid = (n_e // _EBLK,)
    wide = pl.BlockSpec((_EBLK, 2 * LATENT_C), lambda i: (i, 0))
    if kind == "first":
        e_spec = pl.BlockSpec((LATENT_C, _EBLK), lambda i: (0, i))
    else:
        e_spec = wide
    full = lambda a: pl.BlockSpec(a.shape, lambda i: (0,) * a.ndim)
    body = {"first": _edge_mlp_first_body, "mid": _edge_mlp_mid_body,
            "last": _edge_mlp_last_body}[kind]
    wide_out = jax.ShapeDtypeStruct((n_e, 2 * LATENT_C), jnp.float32)
    if kind == "last":
        out_specs = [wide,
                     pl.BlockSpec((LATENT_C, _EBLK), lambda i: (0, i))]
        out_shape = [wide_out,
                     jax.ShapeDtypeStruct((LATENT_C, n_e), jnp.float32)]
    else:
        out_specs = wide
        out_shape = wide_out
    w1g = w1[:2 * LATENT_C]
    w1e = w1[2 * LATENT_C:]
    return pl.pallas_call(
        body,
        grid=grid,
        in_specs=[wide, e_spec,
                  full(w1g), full(w1e), full(b1), full(w2), full(b2),
                  full(g), full(b)],
        out_specs=out_specs,
        out_shape=out_shape,
    )(g2, e, w1g, w1e, b1, w2, b2, g, b)


_NBLK = 5000


def _node_mlp_body(x_ref, a_ref, w1x_ref, w1a_ref, b1_ref, w2_ref, b2_ref,
                   g_ref, bb_ref, xnew_ref):
    h = _bdot(x_ref[...], w1x_ref[...])
    h = h + _bdot(a_ref[...], w1a_ref[...])
    h = h + b1_ref[...]
    h = h * jax.nn.sigmoid(h)
    o = _bdot(h, w2_ref[...])
    o = o + b2_ref[...]
    mu = jnp.mean(o, axis=-1, keepdims=True)
    var = jnp.mean((o - mu) ** 2, axis=-1, keepdims=True)
    on = (o - mu) * lax.rsqrt(var + 1e-5)
    xnew_ref[...] = x_ref[...] + on * g_ref[...] + bb_ref[...]


def _tc_node_mlp(x, agg, w1, b1, w2, b2, g, b):
    grid = (N_NODES_C // _NBLK,)
    row_spec = pl.BlockSpec((_NBLK, LATENT_C), lambda i: (i, 0))
    full = lambda a: pl.BlockSpec(a.shape, lambda i: (0,) * a.ndim)
    w1x = w1[:LATENT_C]
    w1a = w1[LATENT_C:]
    return pl.pallas_call(
        _node_mlp_body,
        grid=grid,
        in_specs=[row_spec, row_spec,
                  full(w1x), full(w1a), full(b1), full(w2), full(b2),
                  full(g), full(b)],
        out_specs=row_spec,
        out_shape=jax.ShapeDtypeStruct((N_NODES_C, LATENT_C), jnp.float32),
    )(x, agg, w1x, w1a, b1, w2, b2, g, b)


def kernel(x, edge_index, edge_attr, params):
    src2 = edge_index[0].astype(jnp.int32).reshape(N_EDGES_C // _IDXW, _IDXW)
    dst2 = edge_index[1].astype(jnp.int32).reshape(N_EDGES_C // _IDXW, _IDXW)
    row2 = lambda a: a.reshape(1, -1)
    n_layers = len(params)
    prev2 = None
    e_new = None
    for li, lp in enumerate(params):
        ep, np_ = lp['edge'], lp['node']
        kind = ("first" if li == 0 else
                "last" if li == n_layers - 1 else "mid")
        g2 = _sc_gather_kernel(N_EDGES_C)(x, src2, dst2)
        e_arg = edge_attr.T if li == 0 else prev2
        res = _tc_edge_mlp(kind, g2, e_arg,
                           ep['W1'], row2(ep['b1']),
                           ep['W2'], row2(ep['b2']),
                           row2(ep['g']), row2(ep['b']))
        if kind == "last":
            upd2, enew_t = res
            e_new = enew_t.T
        else:
            upd2 = res
        agg = _sc_scatter_kernel(N_EDGES_C)(upd2, dst2)
        x = _tc_node_mlp(x, agg,
                         np_['W1'], row2(np_['b1']),
                         np_['W2'], row2(np_['b2']),
                         row2(np_['g']), row2(np_['b']))
        prev2 = upd2
    return (x, e_new)


# speedup vs baseline: 1.0583x; 1.0128x over previous
import functools

import jax
import jax.numpy as jnp
from jax import lax
from jax.experimental import pallas as pl
from jax.experimental.pallas import tpu as pltpu
from jax.experimental.pallas import tpu_sc as plsc

N_NODES_C = 50000
N_EDGES_C = 800000
LATENT_C = 64

_NC = 2
_NS = 16
_NW = _NC * _NS

_IDXW = 125


@functools.lru_cache(maxsize=None)
def _sc_mesh():
    return plsc.VectorSubcoreMesh(core_axis_name="c", subcore_axis_name="s",
                                  num_cores=_NC, num_subcores=_NS)


_G = 250


@functools.lru_cache(maxsize=None)
def _sc_gather_kernel(n_edges):
    e_w = n_edges // _NW
    ng = e_w // _G
    nrow = e_w // _IDXW

    def body(x_hbm, src_hbm, dst_hbm, g2_hbm,
             sidx, didx, srows, drows, gsem, dsem, osem_s, osem_r):
        wid = lax.axis_index("s") * _NC + lax.axis_index("c")
        e0 = wid * e_w
        pltpu.sync_copy(src_hbm.at[pl.ds(wid * nrow, nrow)], sidx)
        pltpu.sync_copy(dst_hbm.at[pl.ds(wid * nrow, nrow)], didx)

        def fire(g, b):
            for j in range(_G // _IDXW):
                kk = g * (_G // _IDXW) + j
                pltpu.async_copy(x_hbm.at[sidx.at[kk]],
                                 srows[b].at[pl.ds(j * _IDXW, _IDXW)],
                                 gsem[b])
                pltpu.async_copy(x_hbm.at[didx.at[kk]],
                                 drows[b].at[pl.ds(j * _IDXW, _IDXW)],
                                 dsem[b])

        def drain_gathers(b):
            for j in range(_G // _IDXW):
                pltpu.make_async_copy(
                    x_hbm.at[sidx.at[0]],
                    srows[b].at[pl.ds(0, _IDXW)], gsem[b]).wait()
                pltpu.make_async_copy(
                    x_hbm.at[didx.at[0]],
                    drows[b].at[pl.ds(0, _IDXW)], dsem[b]).wait()

        def writeback(g, b):
            base = e0 + g * _G
            pltpu.async_copy(
                srows[b], g2_hbm.at[pl.ds(base, _G), pl.ds(0, LATENT_C)],
                osem_s[b])
            pltpu.async_copy(
                drows[b],
                g2_hbm.at[pl.ds(base, _G), pl.ds(LATENT_C, LATENT_C)],
                osem_r[b])

        def wait_writeback(b):
            pltpu.make_async_copy(
                srows[b], g2_hbm.at[pl.ds(0, _G), pl.ds(0, LATENT_C)],
                osem_s[b]).wait()
            pltpu.make_async_copy(
                drows[b], g2_hbm.at[pl.ds(0, _G), pl.ds(LATENT_C, LATENT_C)],
                osem_r[b]).wait()

        def loop(i, _):
            for b in range(2):
                g = 2 * i + b

                @pl.when(g >= 2)
                def _():
                    wait_writeback(b)

                fire(g, b)

                @pl.when(g >= 1)
                def _():
                    drain_gathers(1 - b)
                    writeback(g - 1, 1 - b)

            return ()

        lax.fori_loop(0, ng // 2, loop, ())
        drain_gathers(1)
        writeback(ng - 1, 1)
        wait_writeback(0)
        wait_writeback(1)

    return functools.partial(
        pl.kernel,
        out_type=jax.ShapeDtypeStruct((n_edges, 2 * LATENT_C), jnp.float32),
        mesh=_sc_mesh(),
        scratch_types=[
            pltpu.VMEM((nrow, _IDXW), jnp.int32),
            pltpu.VMEM((nrow, _IDXW), jnp.int32),
            [pltpu.VMEM((_G, LATENT_C), jnp.float32) for _ in range(2)],
            [pltpu.VMEM((_G, LATENT_C), jnp.float32) for _ in range(2)],
            [pltpu.SemaphoreType.DMA for _ in range(2)],
            [pltpu.SemaphoreType.DMA for _ in range(2)],
            [pltpu.SemaphoreType.DMA for _ in range(2)],
            [pltpu.SemaphoreType.DMA for _ in range(2)],
        ],
        compiler_params=pltpu.CompilerParams(use_tc_tiling_on_sc=False),
    )(body)


_COLS = LATENT_C // _NC
_G2 = 250
_RPT = N_NODES_C // _NS
_GIR = _G2 // _IDXW


@functools.lru_cache(maxsize=None)
def _sc_scatter_kernel(n_edges):
    e_t = n_edges // _NS
    ng2 = e_t // _G2

    def body(upd_hbm, dst_hbm, agg_hbm, idx2, rows, rsem, isem, acc_sh):
        c = lax.axis_index("c")
        s = lax.axis_index("s")
        col0 = c * _COLS
        r0 = s * _RPT
        e0 = s * e_t
        i0 = s * (e_t // _IDXW)

        zero16 = jnp.zeros((16,), jnp.float32)

        def zfill(i, _):
            rows[0][i // 2, pl.ds((i % 2) * 16, 16)] = zero16
            return ()

        lax.fori_loop(0, _G2 * 2, zfill, ())

        nfull = _RPT // _G2
        rem = _RPT - nfull * _G2

        def zcopy(kk, _):
            pltpu.sync_copy(rows[0], acc_sh.at[pl.ds(r0 + kk * _G2, _G2)])
            return ()

        lax.fori_loop(0, nfull, zcopy, ())
        pltpu.sync_copy(rows[0].at[pl.ds(0, rem)],
                        acc_sh.at[pl.ds(r0 + nfull * _G2, rem)])
        plsc.subcore_barrier()

        def fire(g, b):
            base = e0 + g * _G2
            pltpu.async_copy(dst_hbm.at[pl.ds(i0 + g * _GIR, _GIR)],
                             idx2[b], isem[b])
            pltpu.async_copy(
                upd_hbm.at[pl.ds(base, _G2), pl.ds(col0, _COLS)],
                rows[b], rsem[b])

        def complete(b):
            pltpu.make_async_copy(dst_hbm.at[pl.ds(0, _GIR)],
                                  idx2[b], isem[b]).wait()
            pltpu.make_async_copy(
                upd_hbm.at[pl.ds(0, _G2), pl.ds(col0, _COLS)],
                rows[b], rsem[b]).wait()
            for j in range(_GIR):
                pltpu.sync_copy(rows[b].at[pl.ds(j * _IDXW, _IDXW)],
                                acc_sh.at[idx2[b].at[j]], add=True)

        def loop(i, _):
            for b in range(2):
                g = 2 * i + b
                fire(g, b)

                @pl.when(g >= 1)
                def _():
                    complete(1 - b)

            return ()

        lax.fori_loop(0, ng2 // 2, loop, ())
        complete(1)
        plsc.subcore_barrier()

        pltpu.sync_copy(acc_sh.at[pl.ds(r0, _RPT)],
                        agg_hbm.at[pl.ds(r0, _RPT), pl.ds(col0, _COLS)])

    return functools.partial(
        pl.kernel,
        out_type=jax.ShapeDtypeStruct((N_NODES_C, LATENT_C), jnp.float32),
        mesh=_sc_mesh(),
        scratch_types=[
            [pltpu.VMEM((_GIR, _IDXW), jnp.int32) for _ in range(2)],
            [pltpu.VMEM((_G2, _COLS), jnp.float32) for _ in range(2)],
            [pltpu.SemaphoreType.DMA for _ in range(2)],
            [pltpu.SemaphoreType.DMA for _ in range(2)],
            pltpu.VMEM_SHARED((N_NODES_C, _COLS), jnp.float32),
        ],
        compiler_params=pltpu.CompilerParams(use_tc_tiling_on_sc=False),
    )(body)


_EBLK = 10000
_EBLK_T = 6400


def _bdot(a, b):
    return jnp.dot(a, b, preferred_element_type=jnp.float32)


def _edge_core(g2b, e, w1g_ref, w1e_ref, b1_ref, w2_ref, b2_ref,
               g_ref, bb_ref, last):
    h = _bdot(g2b, w1g_ref[...])
    h = h + _bdot(e, w1e_ref[...])
    h = h + b1_ref[...]
    h = h * jax.nn.sigmoid(h)
    o = _bdot(h, w2_ref[...])
    o = o + b2_ref[...]
    mu = jnp.mean(o, axis=-1, keepdims=True)
    var = jnp.mean((o - mu) ** 2, axis=-1, keepdims=True)
    on = (o - mu) * lax.rsqrt(var + 1e-5)
    upd = on * g_ref[...] + bb_ref[...]
    if last:
        return upd, e + upd
    return jnp.concatenate([upd, e + upd], axis=-1), None


def _edge_mlp_first_body(g2_ref, e_ref, w1g_ref, w1e_ref, b1_ref, w2_ref,
                         b2_ref, g_ref, bb_ref, upd2_ref):
    e = e_ref[...].T
    upd2_ref[...], _ = _edge_core(g2_ref[...], e, w1g_ref, w1e_ref,
                                  b1_ref, w2_ref, b2_ref, g_ref, bb_ref,
                                  last=False)


def _edge_mlp_mid_body(g2_ref, p2_ref, w1g_ref, w1e_ref, b1_ref, w2_ref,
                       b2_ref, g_ref, bb_ref, upd2_ref):
    upd2_ref[...], _ = _edge_core(g2_ref[...], p2_ref[:, LATENT_C:],
                                  w1g_ref, w1e_ref, b1_ref, w2_ref, b2_ref,
                                  g_ref, bb_ref, last=False)


def _edge_mlp_last_body(g2_ref, p2_ref, w1g_ref, w1e_ref, b1_ref, w2_ref,
                        b2_ref, g_ref, bb_ref, upd2_ref, enew_ref):
    upd, enew = _edge_core(g2_ref[...], p2_ref[:, LATENT_C:],
                           w1g_ref, w1e_ref, b1_ref, w2_ref, b2_ref,
                           g_ref, bb_ref, last=True)
    upd2_ref[...] = jnp.concatenate([upd, upd], axis=-1)
    enew_ref[...] = enew.T


def _tc_edge_mlp(kind, g2, e, w1, b1, w2, b2, g, b):
    n_e = g2.shape[0]
    blk = _EBLK if kind == "mid" else _EBLK_T
    grid = (n_e // blk,)
    wide = pl.BlockSpec((blk, 2 * LATENT_C), lambda i: (i, 0))
    if kind == "first":
        e_spec = pl.BlockSpec((LATENT_C, blk), lambda i: (0, i))
    else:
        e_spec = wide
    full = lambda a: pl.BlockSpec(a.shape, lambda i: (0,) * a.ndim)
    body = {"first": _edge_mlp_first_body, "mid": _edge_mlp_mid_body,
            "last": _edge_mlp_last_body}[kind]
    wide_out = jax.ShapeDtypeStruct((n_e, 2 * LATENT_C), jnp.float32)
    if kind == "last":
        out_specs = [wide,
                     pl.BlockSpec((LATENT_C, blk), lambda i: (0, i))]
        out_shape = [wide_out,
                     jax.ShapeDtypeStruct((LATENT_C, n_e), jnp.float32)]
    else:
        out_specs = wide
        out_shape = wide_out
    w1g = w1[:2 * LATENT_C]
    w1e = w1[2 * LATENT_C:]
    return pl.pallas_call(
        body,
        grid=grid,
        in_specs=[wide, e_spec,
                  full(w1g), full(w1e), full(b1), full(w2), full(b2),
                  full(g), full(b)],
        out_specs=out_specs,
        out_shape=out_shape,
    )(g2, e, w1g, w1e, b1, w2, b2, g, b)


_NBLK = 5000


def _node_mlp_body(x_ref, a_ref, w1x_ref, w1a_ref, b1_ref, w2_ref, b2_ref,
                   g_ref, bb_ref, xnew_ref):
    h = _bdot(x_ref[...], w1x_ref[...])
    h = h + _bdot(a_ref[...], w1a_ref[...])
    h = h + b1_ref[...]
    h = h * jax.nn.sigmoid(h)
    o = _bdot(h, w2_ref[...])
    o = o + b2_ref[...]
    mu = jnp.mean(o, axis=-1, keepdims=True)
    var = jnp.mean((o - mu) ** 2, axis=-1, keepdims=True)
    on = (o - mu) * lax.rsqrt(var + 1e-5)
    xnew_ref[...] = x_ref[...] + on * g_ref[...] + bb_ref[...]


def _tc_node_mlp(x, agg, w1, b1, w2, b2, g, b):
    grid = (N_NODES_C // _NBLK,)
    row_spec = pl.BlockSpec((_NBLK, LATENT_C), lambda i: (i, 0))
    full = lambda a: pl.BlockSpec(a.shape, lambda i: (0,) * a.ndim)
    w1x = w1[:LATENT_C]
    w1a = w1[LATENT_C:]
    return pl.pallas_call(
        _node_mlp_body,
        grid=grid,
        in_specs=[row_spec, row_spec,
                  full(w1x), full(w1a), full(b1), full(w2), full(b2),
                  full(g), full(b)],
        out_specs=row_spec,
        out_shape=jax.ShapeDtypeStruct((N_NODES_C, LATENT_C), jnp.float32),
    )(x, agg, w1x, w1a, b1, w2, b2, g, b)


def kernel(x, edge_index, edge_attr, params):
    src2 = edge_index[0].astype(jnp.int32).reshape(N_EDGES_C // _IDXW, _IDXW)
    dst2 = edge_index[1].astype(jnp.int32).reshape(N_EDGES_C // _IDXW, _IDXW)
    row2 = lambda a: a.reshape(1, -1)
    n_layers = len(params)
    prev2 = None
    e_new = None
    for li, lp in enumerate(params):
        ep, np_ = lp['edge'], lp['node']
        kind = ("first" if li == 0 else
                "last" if li == n_layers - 1 else "mid")
        g2 = _sc_gather_kernel(N_EDGES_C)(x, src2, dst2)
        e_arg = edge_attr.T if li == 0 else prev2
        res = _tc_edge_mlp(kind, g2, e_arg,
                           ep['W1'], row2(ep['b1']),
                           ep['W2'], row2(ep['b2']),
                           row2(ep['g']), row2(ep['b']))
        if kind == "last":
            upd2, enew_t = res
            e_new = enew_t.T
        else:
            upd2 = res
        agg = _sc_scatter_kernel(N_EDGES_C)(upd2, dst2)
        x = _tc_node_mlp(x, agg,
                         np_['W1'], row2(np_['b1']),
                         np_['W2'], row2(np_['b2']),
                         row2(np_['g']), row2(np_['b']))
        prev2 = upd2
    return (x, e_new)
